# baseline (speedup 1.0000x reference)
import functools

import jax
import jax.numpy as jnp
from jax import lax
from jax.experimental import pallas as pl
from jax.experimental.pallas import tpu as pltpu
from jax.experimental.pallas import tpu_sc as plsc

_NC = 2
_NS = 16
_L = 16
_NW = _NC * _NS


def _make_kernel(N, D, E, K, NBUF):
    PW = E // _NW
    NCHUNK = PW // K
    NACC = 1

    mesh = plsc.VectorSubcoreMesh(core_axis_name="c", subcore_axis_name="s")

    scratch = [
        pltpu.VMEM((PW,), jnp.int32),
        pltpu.VMEM((PW,), jnp.int32),
        pltpu.VMEM((PW,), jnp.float32),
    ]
    scratch += [pltpu.VMEM((K, D), jnp.float32) for _ in range(2 * NBUF)]
    scratch += [pltpu.SemaphoreType.DMA for _ in range(NBUF)]

    @functools.partial(
        pl.kernel,
        out_type=jax.ShapeDtypeStruct((E,), jnp.float32),
        mesh=mesh,
        scratch_types=scratch,
        compiler_params=pltpu.CompilerParams(needs_layout_passes=False),
    )
    def ip_decoder(z_hbm, src_hbm, dst_hbm, out_hbm, sidx_v, didx_v, out_v,
                   *bufs):
        srow = bufs[0:NBUF]
        drow = bufs[NBUF:2 * NBUF]
        sems = bufs[2 * NBUF:3 * NBUF]
        wid = lax.axis_index("s") * _NC + lax.axis_index("c")
        base = wid * PW

        pltpu.sync_copy(src_hbm.at[pl.ds(base, PW)], sidx_v)
        pltpu.sync_copy(dst_hbm.at[pl.ds(base, PW)], didx_v)

        def fire(c, j):
            pltpu.async_copy(z_hbm.at[sidx_v.at[pl.ds(c * K, K)]], srow[j], sems[j])
            return pltpu.async_copy(z_hbm.at[didx_v.at[pl.ds(c * K, K)]], drow[j], sems[j])

        def drain(c, j):
            pltpu.make_async_copy(z_hbm.at[sidx_v.at[pl.ds(c * K, K)]], srow[j], sems[j]).wait()
            pltpu.make_async_copy(z_hbm.at[didx_v.at[pl.ds(c * K, K)]], drow[j], sems[j]).wait()

        def compute(c, j):
            lanes = lax.iota(jnp.int32, _L)
            for g in range(K // _L):
                rows = lanes + (g * _L)

                @plsc.parallel_loop(0, _L, 1, unroll=1,
                                    carry=jnp.zeros((_L,), jnp.float32))
                def tstep(t, acc):
                    colb = jnp.bitwise_and(lanes + t, _L - 1)
                    for blk in range(0, D // _L, 2):
                        c0 = colb + (blk * _L)
                        c1 = colb + ((blk + 1) * _L)
                        s0 = plsc.load_gather(srow[j], [rows, c0])
                        d0 = plsc.load_gather(drow[j], [rows, c0])
                        s1 = plsc.load_gather(srow[j], [rows, c1])
                        d1 = plsc.load_gather(drow[j], [rows, c1])
                        acc = acc + (s0 * d0 + s1 * d1)
                    return acc

                res = 1.0 / (1.0 + jnp.exp(-tstep))
                out_v[pl.ds(c * K + g * _L, _L)] = res

        for j in range(NBUF):
            fire(j, j)

        NITER = NCHUNK // NBUF

        def ring_body(i, carry):
            for j in range(NBUF):
                c = i * NBUF + j
                drain(c, j)
                compute(c, j)

                @pl.when(c + NBUF < NCHUNK)
                def _():
                    fire(c + NBUF, j)
            return carry

        lax.fori_loop(0, NITER, ring_body, 0)

        rem = NCHUNK - NITER * NBUF
        for r in range(rem):
            c = NITER * NBUF + r
            j = c % NBUF
            drain(c, j)
            compute(c, j)

        pltpu.sync_copy(out_v, out_hbm.at[pl.ds(wid * PW, PW)])

    return ip_decoder


def kernel(z, edge_index):
    N, D = z.shape
    E = edge_index.shape[1]
    K = 80
    src = edge_index[0]
    dst = edge_index[1]
    fn = _make_kernel(N, D, E, K, 2)
    return fn(z, src, dst)


# speedup vs baseline: 1.2248x; 1.2248x over previous
import functools

import jax
import jax.numpy as jnp
from jax import lax
from jax.experimental import pallas as pl
from jax.experimental.pallas import tpu as pltpu
from jax.experimental.pallas import tpu_sc as plsc

_NC = 2
_NS = 16
_L = 16
_NW = _NC * _NS


def _make_kernel(N, D, E, K, NBUF):
    PW = E // _NW
    NCHUNK = PW // K

    mesh = plsc.VectorSubcoreMesh(core_axis_name="c", subcore_axis_name="s")

    scratch = [
        pltpu.VMEM((PW,), jnp.int32),
        pltpu.VMEM((PW,), jnp.int32),
    ]
    scratch += [pltpu.VMEM((K, D), jnp.float32) for _ in range(2 * NBUF)]
    scratch += [pltpu.VMEM((K,), jnp.float32) for _ in range(NBUF)]
    scratch += [pltpu.SemaphoreType.DMA for _ in range(NBUF)]
    scratch += [pltpu.SemaphoreType.DMA for _ in range(NBUF)]
    scratch += [pltpu.SemaphoreType.DMA, pltpu.SemaphoreType.DMA]

    @functools.partial(
        pl.kernel,
        out_type=jax.ShapeDtypeStruct((E,), jnp.float32),
        mesh=mesh,
        scratch_types=scratch,
        compiler_params=pltpu.CompilerParams(needs_layout_passes=False),
    )
    def ip_decoder(z_hbm, src_hbm, dst_hbm, out_hbm, sidx_v, didx_v, *bufs):
        srow = bufs[0:NBUF]
        drow = bufs[NBUF:2 * NBUF]
        obuf = bufs[2 * NBUF:3 * NBUF]
        sems = bufs[3 * NBUF:4 * NBUF]
        osems = bufs[4 * NBUF:5 * NBUF]
        isem_s, isem_d = bufs[5 * NBUF], bufs[5 * NBUF + 1]
        wid = lax.axis_index("s") * _NC + lax.axis_index("c")
        base = wid * PW

        pltpu.async_copy(src_hbm.at[pl.ds(base, PW)], sidx_v, isem_s)
        pltpu.async_copy(dst_hbm.at[pl.ds(base, PW)], didx_v, isem_d)
        pltpu.make_async_copy(src_hbm.at[pl.ds(base, PW)], sidx_v, isem_s).wait()
        pltpu.make_async_copy(dst_hbm.at[pl.ds(base, PW)], didx_v, isem_d).wait()

        def fire(c, j):
            pltpu.async_copy(z_hbm.at[sidx_v.at[pl.ds(c * K, K)]], srow[j], sems[j])
            return pltpu.async_copy(z_hbm.at[didx_v.at[pl.ds(c * K, K)]], drow[j], sems[j])

        def drain(c, j):
            pltpu.make_async_copy(z_hbm.at[sidx_v.at[pl.ds(c * K, K)]], srow[j], sems[j]).wait()
            pltpu.make_async_copy(z_hbm.at[didx_v.at[pl.ds(c * K, K)]], drow[j], sems[j]).wait()

        def owait(c, j):
            pltpu.make_async_copy(
                obuf[j], out_hbm.at[pl.ds(base + c * K, K)], osems[j]).wait()

        def compute(c, j):
            lanes = lax.iota(jnp.int32, _L)
            for g in range(K // _L):
                rows = lanes + (g * _L)

                @plsc.parallel_loop(0, _L, 1, unroll=1,
                                    carry=jnp.zeros((_L,), jnp.float32))
                def tstep(t, acc):
                    colb = jnp.bitwise_and(lanes + t, _L - 1)
                    for blk in range(0, D // _L, 2):
                        c0 = colb + (blk * _L)
                        c1 = colb + ((blk + 1) * _L)
                        s0 = plsc.load_gather(srow[j], [rows, c0])
                        d0 = plsc.load_gather(drow[j], [rows, c0])
                        s1 = plsc.load_gather(srow[j], [rows, c1])
                        d1 = plsc.load_gather(drow[j], [rows, c1])
                        acc = acc + (s0 * d0 + s1 * d1)
                    return acc

                res = 1.0 / (1.0 + jnp.exp(-tstep))
                obuf[j][pl.ds(g * _L, _L)] = res
            pltpu.async_copy(obuf[j], out_hbm.at[pl.ds(base + c * K, K)],
                             osems[j])

        for j in range(NBUF):
            fire(j, j)

        NITER = NCHUNK // NBUF

        def ring_body(i, carry):
            for j in range(NBUF):
                c = i * NBUF + j
                drain(c, j)

                @pl.when(c >= NBUF)
                def _():
                    owait(c - NBUF, j)

                compute(c, j)

                @pl.when(c + NBUF < NCHUNK)
                def _():
                    fire(c + NBUF, j)
            return carry

        lax.fori_loop(0, NITER, ring_body, 0)

        rem = NCHUNK - NITER * NBUF
        for r in range(rem):
            c = NITER * NBUF + r
            j = c % NBUF
            drain(c, j)
            if c >= NBUF:
                owait(c - NBUF, j)
            compute(c, j)

        for j in range(NBUF):
            c = NCHUNK - NBUF + j
            owait(c, c % NBUF)

    return ip_decoder


def kernel(z, edge_index):
    N, D = z.shape
    E = edge_index.shape[1]
    K = 80
    src = edge_index[0]
    dst = edge_index[1]
    fn = _make_kernel(N, D, E, K, 4)
    return fn(z, src, dst)
